# Initial kernel scaffold; baseline (speedup 1.0000x reference)
import jax
import jax.numpy as jnp
from jax.experimental import pallas as pl


def kernel(x, W_gcn, b_gcn, W1, b1, W2, b2, Wd, bd, edge_index):
    raise NotImplementedError("write your pallas kernel here")


# speedup vs baseline: 12.9884x; 12.9884x over previous
import functools

import jax
import jax.numpy as jnp
from jax import lax
from jax.experimental import pallas as pl
from jax.experimental.pallas import tpu as pltpu
from jax.experimental.pallas import tpu_sc as plsc

N = 10000
E = 160000
D = 256
H = 128
DH = 128
NS = 16
EPT = E // NS
EPT_DEG = E // (2 * NS)
NB = EPT // 128
TAIL = EPT - NB * 128
HIST_R = 80
ROWS_PT = N // NS


NBINS = HIST_R * 128
CHUNK = NBINS // NS
FULL16 = EPT_DEG // 16


def _deg_body(dst_hbm, out0_hbm, out1_hbm, idx_v, hist_v, tmp_v, acc_v,
              part_sh):
  c = lax.axis_index("c")
  s = lax.axis_index("s")
  zeros = jnp.zeros((16,), jnp.float32)
  ones = jnp.ones((16,), jnp.float32)

  def zbin(k, carry):
    hist_v[pl.ds(k * 16, 16)] = zeros
    return carry
  lax.fori_loop(0, NBINS // 16, zbin, 0)

  tid = s * 2 + c
  pltpu.sync_copy(dst_hbm.at[pl.ds(tid * EPT_DEG, EPT_DEG)],
                  idx_v.at[pl.ds(0, EPT_DEG)])

  def body(i, carry):
    idx = idx_v[pl.ds(i * 16, 16)]
    plsc.addupdate_scatter(hist_v, [idx], ones)
    return carry
  lax.fori_loop(0, FULL16, body, 0)
  tail = idx_v[pl.ds(FULL16 * 16, 16)]
  plsc.addupdate_scatter(hist_v, [tail], ones,
                         mask=lax.iota(jnp.int32, 16) < (EPT_DEG - FULL16 * 16))

  pltpu.sync_copy(hist_v, part_sh.at[s])
  plsc.subcore_barrier()

  def zacc(k, carry):
    acc_v[pl.ds(k * 16, 16)] = zeros
    return carry
  lax.fori_loop(0, CHUNK // 16, zacc, 0)

  def merge(t, carry):
    pltpu.sync_copy(part_sh.at[t, pl.ds(s * CHUNK, CHUNK)], tmp_v)

    def addk(k, carry2):
      sl = pl.ds(k * 16, 16)
      acc_v[sl] = acc_v[sl] + tmp_v[sl]
      return carry2
    lax.fori_loop(0, CHUNK // 16, addk, 0)
    return carry
  lax.fori_loop(0, NS, merge, 0)

  @pl.when(c == 0)
  def _():
    pltpu.sync_copy(acc_v, out0_hbm.at[pl.ds(s * CHUNK, CHUNK)])

  @pl.when(c == 1)
  def _():
    pltpu.sync_copy(acc_v, out1_hbm.at[pl.ds(s * CHUNK, CHUNK)])


_deg_kernel = functools.partial(
    pl.kernel,
    out_type=[jax.ShapeDtypeStruct((NBINS,), jnp.float32),
              jax.ShapeDtypeStruct((NBINS,), jnp.float32)],
    mesh=plsc.VectorSubcoreMesh(core_axis_name="c", subcore_axis_name="s"),
    compiler_params=pltpu.CompilerParams(needs_layout_passes=False),
    scratch_types=[
        pltpu.VMEM((EPT_DEG + 8,), jnp.int32),
        pltpu.VMEM((NBINS,), jnp.float32),
        pltpu.VMEM((CHUNK,), jnp.float32),
        pltpu.VMEM((CHUNK,), jnp.float32),
        pltpu.VMEM_SHARED((NS, NBINS), jnp.float32),
    ],
)(_deg_body)


def _seg_body(xs_lo_hbm, xs_hi_hbm, src_hbm, dst_hbm, out_lo, out_hi,
              idx_s, idx_d, rows, idx_st, idx_dt, rows_t, sem, acc_sh):
  c = lax.axis_index("c")
  s = lax.axis_index("s")

  def zrow(k, carry):
    r = k // 8
    c0 = (k % 8) * 16
    rows[r, pl.ds(c0, 16)] = jnp.zeros((16,), jnp.float32)
    return carry
  lax.fori_loop(0, 128 * 8, zrow, 0)

  def zacc(j, carry):
    pltpu.sync_copy(rows, acc_sh.at[pl.ds((s * 5 + j) * 128, 128)])
    return carry
  lax.fori_loop(0, 5, zacc, 0)
  plsc.subcore_barrier()

  e0 = s * EPT

  def run(xs_hbm, out_hbm):
    def blk(i, carry):
      off = e0 + i * 128
      pltpu.sync_copy(src_hbm.at[pl.ds(off, 128)], idx_s)
      pltpu.async_copy(xs_hbm.at[idx_s], rows, sem).wait()
      pltpu.sync_copy(dst_hbm.at[pl.ds(off, 128)], idx_d)
      pltpu.sync_copy(rows, acc_sh.at[idx_d], add=True)
      return carry
    lax.fori_loop(0, NB, blk, 0)

    off = e0 + NB * 128
    pltpu.sync_copy(src_hbm.at[pl.ds(off, TAIL)], idx_st)
    pltpu.async_copy(xs_hbm.at[idx_st], rows_t, sem).wait()
    pltpu.sync_copy(dst_hbm.at[pl.ds(off, TAIL)], idx_dt)
    pltpu.sync_copy(rows_t, acc_sh.at[idx_dt], add=True)

    plsc.subcore_barrier()
    pltpu.sync_copy(acc_sh.at[pl.ds(s * 640, 640)],
                    out_hbm.at[pl.ds(s * 640, 640)])

  @pl.when(c == 0)
  def _():
    run(xs_lo_hbm, out_lo)

  @pl.when(c == 1)
  def _():
    run(xs_hi_hbm, out_hi)


_seg_kernel = functools.partial(
    pl.kernel,
    out_type=[jax.ShapeDtypeStruct((HIST_R * 128, DH), jnp.float32),
              jax.ShapeDtypeStruct((HIST_R * 128, DH), jnp.float32)],
    mesh=plsc.VectorSubcoreMesh(core_axis_name="c", subcore_axis_name="s"),
    scratch_types=[
        pltpu.VMEM((128,), jnp.int32),
        pltpu.VMEM((128,), jnp.int32),
        pltpu.VMEM((128, DH), jnp.float32),
        pltpu.VMEM((TAIL,), jnp.int32),
        pltpu.VMEM((TAIL,), jnp.int32),
        pltpu.VMEM((TAIL, DH), jnp.float32),
        pltpu.SemaphoreType.DMA,
        pltpu.VMEM_SHARED((HIST_R * 128, DH), jnp.float32),
    ],
)(_seg_body)


RB = 1000


def _scale_mm_body(x_ref, w_ref, deg_ref, lo_ref, hi_ref, dinv_ref):
  dinv = lax.rsqrt(deg_ref[...] + 1.0)
  xw = jnp.dot(x_ref[...], w_ref[...], preferred_element_type=jnp.float32)
  xs = xw * dinv
  lo_ref[...] = xs[:, :DH]
  hi_ref[...] = xs[:, DH:]
  dinv_ref[...] = dinv


def _scale_mm(x, w, deg0):
  grid = (N // RB,)
  return pl.pallas_call(
      _scale_mm_body,
      grid=grid,
      in_specs=[
          pl.BlockSpec((RB, D), lambda i: (i, 0)),
          pl.BlockSpec((D, D), lambda i: (0, 0)),
          pl.BlockSpec((RB, 1), lambda i: (i, 0)),
      ],
      out_specs=[
          pl.BlockSpec((RB, DH), lambda i: (i, 0)),
          pl.BlockSpec((RB, DH), lambda i: (i, 0)),
          pl.BlockSpec((RB, 1), lambda i: (i, 0)),
      ],
      out_shape=[
          jax.ShapeDtypeStruct((N, DH), jnp.float32),
          jax.ShapeDtypeStruct((N, DH), jnp.float32),
          jax.ShapeDtypeStruct((N, 1), jnp.float32),
      ],
  )(x, w, deg0)


def _mlp_body(x_ref, xslo_ref, xshi_ref, acclo_ref, acchi_ref, dinv_ref,
              bgcn_ref, w1_ref, b1_ref, w2_ref, b2_ref, wd_ref, bd_ref,
              out_ref):
  dinv = dinv_ref[...]
  bg = bgcn_ref[...]
  conv_lo = dinv * (acclo_ref[...] + xslo_ref[...]) + bg[:, :DH]
  conv_hi = dinv * (acchi_ref[...] + xshi_ref[...]) + bg[:, DH:]
  x = x_ref[...]
  h_lo = jnp.maximum(conv_lo, 0.0) + x[:, :DH]
  h_hi = jnp.maximum(conv_hi, 0.0) + x[:, DH:]
  w1 = w1_ref[...]
  t = (jnp.dot(h_lo, w1[:DH, :], preferred_element_type=jnp.float32)
       + jnp.dot(h_hi, w1[DH:, :], preferred_element_type=jnp.float32)
       + b1_ref[...])
  t = jnp.where(t > 0, t, 0.01 * t)
  t = jnp.dot(t, w2_ref[...], preferred_element_type=jnp.float32) + b2_ref[...]
  t = jnp.where(t > 0, t, 0.01 * t)
  a = jnp.dot(t, wd_ref[...], preferred_element_type=jnp.float32) + bd_ref[...]
  out_ref[...] = jnp.maximum(a, 0.0) + jnp.log1p(jnp.exp(-jnp.abs(a))) + 1e-20


def _mlp(x, xs_lo, xs_hi, acc_lo, acc_hi, dinv, b_gcn, w1, b1, w2, b2, wd, bd):
  grid = (N // RB,)
  full = lambda r, c: pl.BlockSpec((r, c), lambda i: (0, 0))
  return pl.pallas_call(
      _mlp_body,
      grid=grid,
      in_specs=[
          pl.BlockSpec((RB, D), lambda i: (i, 0)),
          pl.BlockSpec((RB, DH), lambda i: (i, 0)),
          pl.BlockSpec((RB, DH), lambda i: (i, 0)),
          pl.BlockSpec((RB, DH), lambda i: (i, 0)),
          pl.BlockSpec((RB, DH), lambda i: (i, 0)),
          pl.BlockSpec((RB, 1), lambda i: (i, 0)),
          full(1, D),
          full(D, H),
          full(1, H),
          full(H, H),
          full(1, H),
          full(H, 1),
          full(1, 1),
      ],
      out_specs=pl.BlockSpec((RB, 1), lambda i: (i, 0)),
      out_shape=jax.ShapeDtypeStruct((N, 1), jnp.float32),
  )(x, xs_lo, xs_hi, acc_lo, acc_hi, dinv, b_gcn, w1, b1, w2, b2, wd, bd)


def kernel(x, W_gcn, b_gcn, W1, b1, W2, b2, Wd, bd, edge_index):
  src = edge_index[0]
  dst = edge_index[1]

  hist0, hist1 = _deg_kernel(dst)
  deg0 = (hist0 + hist1)[:N, None]

  xs_lo, xs_hi, dinv = _scale_mm(x, W_gcn, deg0)

  acc_lo, acc_hi = _seg_kernel(xs_lo, xs_hi, src, dst)

  out = _mlp(x, xs_lo, xs_hi, acc_lo, acc_hi, dinv,
             b_gcn.reshape(1, D), W1, b1.reshape(1, H),
             W2, b2.reshape(1, H), Wd, bd.reshape(1, 1))
  return out.reshape(-1)
